# Initial kernel scaffold; baseline (speedup 1.0000x reference)
import jax
import jax.numpy as jnp
from jax.experimental import pallas as pl


def kernel(dense_features, sharded_sparse_features, tables, d_W1, d_b1, d_W2, d_b2, o_W1, o_b1, o_W2, o_b2):
    raise NotImplementedError("write your pallas kernel here")


# speedup vs baseline: 2.2731x; 2.2731x over previous
import functools

import jax
import jax.numpy as jnp
import numpy as np
from jax import lax
from jax.experimental import pallas as pl
from jax.experimental.pallas import tpu as pltpu
from jax.experimental.pallas import tpu_sc as plsc

B, T, L, V, D = 1024, 26, 20, 100000, 128
HID = 512

NC, NS = 2, 16
NW = NC * NS
NBAGS = B * T
BAGS_PER_W = NBAGS // NW
GROUP_BAGS = 32
GROUPS = BAGS_PER_W // GROUP_BAGS
ROWS_PER_GROUP = GROUP_BAGS * L
CHUNK = 128
CHUNKS_PER_GROUP = ROWS_PER_GROUP // CHUNK
CHUNKS_PER_W = GROUPS * CHUNKS_PER_GROUP
TOTAL_ROWS = NBAGS * L
TOTAL_CHUNKS = TOTAL_ROWS // CHUNK

_SIDX = ((np.arange(ROWS_PER_GROUP, dtype=np.int32) // L).reshape(
    1, CHUNKS_PER_GROUP, CHUNK)
         + (np.arange(NS, dtype=np.int32) * GROUP_BAGS)[:, None, None]
         ).reshape(NS * CHUNKS_PER_GROUP, 1, CHUNK)


def _sc_embedding_bags(table_flat, idx, sidx, zeros_blk):
  mesh = plsc.VectorSubcoreMesh(core_axis_name="c", subcore_axis_name="s")

  @functools.partial(
      pl.kernel,
      out_type=jax.ShapeDtypeStruct((NBAGS, D), jnp.float32),
      mesh=mesh,
      scratch_types=[
          pltpu.VMEM((CHUNKS_PER_GROUP, 1, CHUNK), jnp.int32),
          pltpu.VMEM((CHUNKS_PER_GROUP, 1, CHUNK), jnp.int32),
          pltpu.VMEM((CHUNK, D), jnp.float32),
          pltpu.VMEM((GROUP_BAGS, D), jnp.float32),
          pltpu.VMEM_SHARED((NS * GROUP_BAGS, D), jnp.float32),
      ],
  )
  def kern(table_hbm, idx_hbm, sidx_hbm, zeros_hbm, out_hbm,
           idx_v, sidx_v, rows_v, zero_v, acc_sh):
    sid = lax.axis_index("s")
    wid = lax.axis_index("c") * NS + sid
    pltpu.sync_copy(sidx_hbm.at[pl.ds(sid * CHUNKS_PER_GROUP,
                                      CHUNKS_PER_GROUP)], sidx_v)
    pltpu.sync_copy(zeros_hbm, zero_v)
    abase = sid * GROUP_BAGS

    @pl.loop(0, GROUPS)
    def _(g):
      cbase = wid * CHUNKS_PER_W + g * CHUNKS_PER_GROUP
      pltpu.sync_copy(idx_hbm.at[pl.ds(cbase, CHUNKS_PER_GROUP)], idx_v)
      pltpu.sync_copy(zero_v, acc_sh.at[pl.ds(abase, GROUP_BAGS)])
      for k in range(CHUNKS_PER_GROUP):
        pltpu.sync_copy(table_hbm.at[idx_v.at[k, 0]], rows_v)
        pltpu.sync_copy(rows_v, acc_sh.at[sidx_v.at[k, 0]], add=True)
      obase = wid * BAGS_PER_W + g * GROUP_BAGS
      pltpu.sync_copy(acc_sh.at[pl.ds(abase, GROUP_BAGS)],
                      out_hbm.at[pl.ds(obase, GROUP_BAGS)])

  return kern(table_flat, idx, sidx, zeros_blk)


def _dense_mlp_body(x_ref, w1_ref, b1_ref, w2_ref, b2_ref, o_ref):
  h = jnp.maximum(jnp.dot(x_ref[...], w1_ref[...],
                          preferred_element_type=jnp.float32) + b1_ref[...],
                  0.0)
  o = jnp.dot(h, w2_ref[...], preferred_element_type=jnp.float32) + b2_ref[...]
  o_ref[...] = jnp.maximum(o, 0.0)


def _dense_mlp(x, w1, b1, w2, b2):
  return pl.pallas_call(
      _dense_mlp_body,
      out_shape=jax.ShapeDtypeStruct((B, D), jnp.float32),
  )(x, w1, b1.reshape(1, -1), w2, b2.reshape(1, -1))


def _top_mlp_body(dx_ref, emb_ref, w1a_ref, w1b_ref, b1_ref, w2_ref, b2_ref,
                  o_ref):
  h = jnp.dot(dx_ref[...], w1a_ref[...], preferred_element_type=jnp.float32)
  h += jnp.dot(emb_ref[...], w1b_ref[...], preferred_element_type=jnp.float32)
  h = jnp.maximum(h + b1_ref[...], 0.0)
  o = jnp.sum(h * w2_ref[...], axis=1, keepdims=True) + b2_ref[...]
  o_ref[...] = jnp.maximum(o, 0.0)


def _top_mlp(dense_x, emb_flat, w1a, w1b, b1, w2_row, b2):
  blk = 256
  grid = (B // blk,)
  return pl.pallas_call(
      _top_mlp_body,
      grid=grid,
      in_specs=[
          pl.BlockSpec((blk, D), lambda i: (i, 0)),
          pl.BlockSpec((blk, T * D), lambda i: (i, 0)),
          pl.BlockSpec((D, HID), lambda i: (0, 0)),
          pl.BlockSpec((T * D, HID), lambda i: (0, 0)),
          pl.BlockSpec((1, HID), lambda i: (0, 0)),
          pl.BlockSpec((1, HID), lambda i: (0, 0)),
          pl.BlockSpec((1, 1), lambda i: (0, 0)),
      ],
      out_specs=pl.BlockSpec((blk, 1), lambda i: (i, 0)),
      out_shape=jax.ShapeDtypeStruct((B, 1), jnp.float32),
  )(dense_x, emb_flat, w1a, w1b, b1.reshape(1, -1), w2_row, b2.reshape(1, 1))


def kernel(dense_features, sharded_sparse_features, tables,
           d_W1, d_b1, d_W2, d_b2, o_W1, o_b1, o_W2, o_b2):
  offs = (jnp.arange(T, dtype=jnp.int32) * V)[None, :, None]
  idx = (sharded_sparse_features.astype(jnp.int32) + offs).reshape(
      TOTAL_CHUNKS, 1, CHUNK)
  table_flat = tables.reshape(T * V, D)
  sidx = jnp.asarray(_SIDX)
  zeros_blk = jnp.zeros((GROUP_BAGS, D), jnp.float32)

  emb = _sc_embedding_bags(table_flat, idx, sidx, zeros_blk)
  dense_x = _dense_mlp(dense_features, d_W1, d_b1, d_W2, d_b2)
  logits = _top_mlp(dense_x, emb.reshape(B, T * D),
                    o_W1[:D], o_W1[D:], o_b1, o_W2.reshape(1, HID), o_b2)
  return logits


# speedup vs baseline: 2.8705x; 1.2628x over previous
import functools

import jax
import jax.numpy as jnp
import numpy as np
from jax import lax
from jax.experimental import pallas as pl
from jax.experimental.pallas import tpu as pltpu
from jax.experimental.pallas import tpu_sc as plsc

B, T, L, V, D = 1024, 26, 20, 100000, 128
HID = 512

NC, NS = 2, 16
NW = NC * NS
NBAGS = B * T
BAGS_PER_W = NBAGS // NW
GROUP_BAGS = 64
GROUPS = BAGS_PER_W // GROUP_BAGS
ROWS_PER_GROUP = GROUP_BAGS * L
CHUNK = 128
CHUNKS_PER_GROUP = ROWS_PER_GROUP // CHUNK
CHUNKS_PER_W = GROUPS * CHUNKS_PER_GROUP
TOTAL_ROWS = NBAGS * L
TOTAL_CHUNKS = TOTAL_ROWS // CHUNK

_SIDX = ((np.arange(ROWS_PER_GROUP, dtype=np.int32) // L).reshape(
    1, CHUNKS_PER_GROUP, CHUNK)
         + (np.arange(NS, dtype=np.int32) * GROUP_BAGS)[:, None, None]
         ).reshape(NS * CHUNKS_PER_GROUP, 1, CHUNK)


def _sc_embedding_bags(table_flat, idx, sidx, zeros_blk):
  mesh = plsc.VectorSubcoreMesh(core_axis_name="c", subcore_axis_name="s")

  @functools.partial(
      pl.kernel,
      out_type=jax.ShapeDtypeStruct((NBAGS, D), jnp.float32),
      mesh=mesh,
      scratch_types=[
          pltpu.VMEM((2, CHUNKS_PER_GROUP, 1, CHUNK), jnp.int32),
          pltpu.VMEM((CHUNKS_PER_GROUP, 1, CHUNK), jnp.int32),
          pltpu.VMEM((2, CHUNK, D), jnp.float32),
          pltpu.VMEM((CHUNK, D), jnp.float32),
          pltpu.VMEM_SHARED((NS * GROUP_BAGS, D), jnp.float32),
          pltpu.SemaphoreType.DMA,
      ],
  )
  def kern(table_hbm, idx_hbm, sidx_hbm, zeros_hbm, out_hbm,
           idx_v, sidx_v, rows_v, zero_v, acc_sh, gsem):
    sid = lax.axis_index("s")
    wid = lax.axis_index("c") * NS + sid
    pltpu.sync_copy(sidx_hbm.at[pl.ds(sid * CHUNKS_PER_GROUP,
                                      CHUNKS_PER_GROUP)], sidx_v)
    pltpu.sync_copy(zeros_hbm, zero_v)
    abase = sid * GROUP_BAGS

    def fire_gather(p, k, buf):
      pltpu.async_copy(table_hbm.at[idx_v.at[p, k, 0]], rows_v.at[buf], gsem)

    def wait_gather(p, k, buf):
      pltpu.make_async_copy(table_hbm.at[idx_v.at[p, k, 0]],
                            rows_v.at[buf], gsem).wait()

    pltpu.sync_copy(idx_hbm.at[pl.ds(wid * CHUNKS_PER_W, CHUNKS_PER_GROUP)],
                    idx_v.at[0])
    pltpu.sync_copy(zero_v.at[pl.ds(0, GROUP_BAGS)],
                    acc_sh.at[pl.ds(abase, GROUP_BAGS)])
    fire_gather(0, 0, 0)

    @pl.loop(0, GROUPS)
    def _(g):
      p = lax.rem(g, 2)
      pn = 1 - p

      @pl.when(g < GROUPS - 1)
      def _():
        cbase = wid * CHUNKS_PER_W + (g + 1) * CHUNKS_PER_GROUP
        pltpu.sync_copy(idx_hbm.at[pl.ds(cbase, CHUNKS_PER_GROUP)],
                        idx_v.at[pn])

      for k in range(CHUNKS_PER_GROUP):
        buf = k % 2
        wait_gather(p, k, buf)
        if k < CHUNKS_PER_GROUP - 1:
          fire_gather(p, k + 1, 1 - buf)
        else:
          @pl.when(g < GROUPS - 1)
          def _():
            fire_gather(pn, 0, 1 - buf)
        pltpu.sync_copy(rows_v.at[buf], acc_sh.at[sidx_v.at[k, 0]], add=True)

      pltpu.sync_copy(zero_v,
                      acc_sh.at[sidx_v.at[CHUNKS_PER_GROUP - 1, 0]], add=True)
      obase = wid * BAGS_PER_W + g * GROUP_BAGS
      pltpu.sync_copy(acc_sh.at[pl.ds(abase, GROUP_BAGS)],
                      out_hbm.at[pl.ds(obase, GROUP_BAGS)])
      pltpu.sync_copy(zero_v.at[pl.ds(0, GROUP_BAGS)],
                      acc_sh.at[pl.ds(abase, GROUP_BAGS)])

  return kern(table_flat, idx, sidx, zeros_blk)


def _dense_mlp_body(x_ref, w1_ref, b1_ref, w2_ref, b2_ref, o_ref):
  h = jnp.maximum(jnp.dot(x_ref[...], w1_ref[...],
                          preferred_element_type=jnp.float32) + b1_ref[...],
                  0.0)
  o = jnp.dot(h, w2_ref[...], preferred_element_type=jnp.float32) + b2_ref[...]
  o_ref[...] = jnp.maximum(o, 0.0).astype(jnp.bfloat16)


def _dense_mlp(x, w1, b1, w2, b2):
  return pl.pallas_call(
      _dense_mlp_body,
      out_shape=jax.ShapeDtypeStruct((B, D), jnp.bfloat16),
  )(x, w1, b1.reshape(1, -1), w2, b2.reshape(1, -1))


def _top_mlp_body(dx_ref, emb_ref, w1a_ref, w1b_ref, b1_ref, w2_ref, b2_ref,
                  o_ref):
  h = jnp.dot(dx_ref[...], w1a_ref[...].astype(jnp.bfloat16),
              preferred_element_type=jnp.float32)
  h += jnp.dot(emb_ref[...], w1b_ref[...].astype(jnp.bfloat16),
               preferred_element_type=jnp.float32)
  h = jnp.maximum(h + b1_ref[...], 0.0)
  o = jnp.dot(h, w2_ref[...], preferred_element_type=jnp.float32)
  o_ref[...] = jnp.maximum(o + b2_ref[...], 0.0)


def _top_mlp(dense_x, emb_flat, w1a, w1b, b1, w2_pad, b2):
  blk = 256
  grid = (B // blk,)
  return pl.pallas_call(
      _top_mlp_body,
      grid=grid,
      in_specs=[
          pl.BlockSpec((blk, D), lambda i: (i, 0)),
          pl.BlockSpec((blk, T * D), lambda i: (i, 0)),
          pl.BlockSpec((D, HID), lambda i: (0, 0)),
          pl.BlockSpec((T * D, HID), lambda i: (0, 0)),
          pl.BlockSpec((1, HID), lambda i: (0, 0)),
          pl.BlockSpec((HID, 128), lambda i: (0, 0)),
          pl.BlockSpec((1, 1), lambda i: (0, 0)),
      ],
      out_specs=pl.BlockSpec((blk, 128), lambda i: (i, 0)),
      out_shape=jax.ShapeDtypeStruct((B, 128), jnp.float32),
  )(dense_x, emb_flat, w1a, w1b, b1.reshape(1, -1), w2_pad, b2.reshape(1, 1))


def kernel(dense_features, sharded_sparse_features, tables,
           d_W1, d_b1, d_W2, d_b2, o_W1, o_b1, o_W2, o_b2):
  offs = (jnp.arange(T, dtype=jnp.int32) * V)[None, :, None]
  idx = (sharded_sparse_features.astype(jnp.int32) + offs).reshape(
      TOTAL_CHUNKS, 1, CHUNK)
  table_flat = tables.reshape(T * V, D)
  sidx = jnp.asarray(_SIDX)
  zeros_blk = jnp.zeros((CHUNK, D), jnp.float32)

  emb = _sc_embedding_bags(table_flat, idx, sidx, zeros_blk)
  dense_x = _dense_mlp(dense_features, d_W1, d_b1, d_W2, d_b2)
  emb_bf = emb.reshape(B, T * D).astype(jnp.bfloat16)
  w2_pad = jnp.pad(o_W2, ((0, 0), (0, 127)))
  logits_pad = _top_mlp(dense_x, emb_bf,
                        o_W1[:D], o_W1[D:], o_b1, w2_pad, o_b2)
  return logits_pad[:, :1]


# speedup vs baseline: 2.9417x; 1.0248x over previous
import functools

import jax
import jax.numpy as jnp
import numpy as np
from jax import lax
from jax.experimental import pallas as pl
from jax.experimental.pallas import tpu as pltpu
from jax.experimental.pallas import tpu_sc as plsc

B, T, L, V, D = 1024, 26, 20, 100000, 128
HID = 512

NC, NS = 2, 16
NW = NC * NS
NBAGS = B * T
BAGS_PER_W = NBAGS // NW
GROUP_BAGS = 64
GROUPS = BAGS_PER_W // GROUP_BAGS
ROWS_PER_GROUP = GROUP_BAGS * L
CHUNK = 128
CHUNKS_PER_GROUP = ROWS_PER_GROUP // CHUNK
CHUNKS_PER_W = GROUPS * CHUNKS_PER_GROUP
TOTAL_ROWS = NBAGS * L
TOTAL_CHUNKS = TOTAL_ROWS // CHUNK

_SIDX = ((np.arange(ROWS_PER_GROUP, dtype=np.int32) // L).reshape(
    1, 1, CHUNKS_PER_GROUP, CHUNK)
         + (np.arange(2, dtype=np.int32) * GROUP_BAGS)[None, :, None, None]
         + (np.arange(NS, dtype=np.int32) * 2 * GROUP_BAGS)[:, None, None,
                                                            None]
         ).reshape(NS * 2 * CHUNKS_PER_GROUP, 1, CHUNK)


def _sc_embedding_bags(table_flat, idx, sidx, zeros_blk):
  mesh = plsc.VectorSubcoreMesh(core_axis_name="c", subcore_axis_name="s")

  @functools.partial(
      pl.kernel,
      out_type=jax.ShapeDtypeStruct((NBAGS, D), jnp.float32),
      mesh=mesh,
      scratch_types=[
          pltpu.VMEM((2, CHUNKS_PER_GROUP, 1, CHUNK), jnp.int32),
          pltpu.VMEM((2 * CHUNKS_PER_GROUP, 1, CHUNK), jnp.int32),
          pltpu.VMEM((2, CHUNK, D), jnp.float32),
          pltpu.VMEM((CHUNK, D), jnp.float32),
          pltpu.VMEM_SHARED((NS * 2 * GROUP_BAGS, D), jnp.float32),
          pltpu.SemaphoreType.DMA,
          pltpu.SemaphoreType.DMA,
      ],
  )
  def kern(table_hbm, idx_hbm, sidx_hbm, zeros_hbm, out_hbm,
           idx_v, sidx_v, rows_v, zero_v, acc_sh, gsem, osem):
    sid = lax.axis_index("s")
    wid = lax.axis_index("c") * NS + sid
    pltpu.sync_copy(sidx_hbm.at[pl.ds(sid * 2 * CHUNKS_PER_GROUP,
                                      2 * CHUNKS_PER_GROUP)], sidx_v)
    pltpu.sync_copy(zeros_hbm, zero_v)
    abase = sid * 2 * GROUP_BAGS

    def fire_gather(p, k, buf):
      pltpu.async_copy(table_hbm.at[idx_v.at[p, k, 0]], rows_v.at[buf], gsem)

    def wait_gather(p, k, buf):
      pltpu.make_async_copy(table_hbm.at[idx_v.at[p, k, 0]],
                            rows_v.at[buf], gsem).wait()

    pltpu.sync_copy(idx_hbm.at[pl.ds(wid * CHUNKS_PER_W, CHUNKS_PER_GROUP)],
                    idx_v.at[0])
    fire_gather(0, 0, 0)

    @pl.loop(0, GROUPS)
    def _(g):
      p = lax.rem(g, 2)
      pn = 1 - p
      rbase = abase + p * GROUP_BAGS

      @pl.when(g >= 2)
      def _():
        pltpu.make_async_copy(acc_sh.at[pl.ds(rbase, GROUP_BAGS)],
                              out_hbm.at[pl.ds(0, GROUP_BAGS)], osem).wait()
      pltpu.sync_copy(zero_v.at[pl.ds(0, GROUP_BAGS)],
                      acc_sh.at[pl.ds(rbase, GROUP_BAGS)])

      @pl.when(g < GROUPS - 1)
      def _():
        cbase = wid * CHUNKS_PER_W + (g + 1) * CHUNKS_PER_GROUP
        pltpu.sync_copy(idx_hbm.at[pl.ds(cbase, CHUNKS_PER_GROUP)],
                        idx_v.at[pn])

      for k in range(CHUNKS_PER_GROUP):
        buf = k % 2
        wait_gather(p, k, buf)
        if k < CHUNKS_PER_GROUP - 1:
          fire_gather(p, k + 1, 1 - buf)
        else:
          @pl.when(g < GROUPS - 1)
          def _():
            fire_gather(pn, 0, 1 - buf)
        pltpu.sync_copy(rows_v.at[buf],
                        acc_sh.at[sidx_v.at[p * CHUNKS_PER_GROUP + k, 0]],
                        add=True)

      pltpu.sync_copy(
          zero_v,
          acc_sh.at[sidx_v.at[(p + 1) * CHUNKS_PER_GROUP - 1, 0]], add=True)
      obase = wid * BAGS_PER_W + g * GROUP_BAGS
      pltpu.async_copy(acc_sh.at[pl.ds(rbase, GROUP_BAGS)],
                       out_hbm.at[pl.ds(obase, GROUP_BAGS)], osem)

    for _ in range(2):
      pltpu.make_async_copy(acc_sh.at[pl.ds(abase, GROUP_BAGS)],
                            out_hbm.at[pl.ds(0, GROUP_BAGS)], osem).wait()

  return kern(table_flat, idx, sidx, zeros_blk)


def _dense_mlp_body(x_ref, w1_ref, b1_ref, w2_ref, b2_ref, o_ref):
  h = jnp.maximum(jnp.dot(x_ref[...], w1_ref[...],
                          preferred_element_type=jnp.float32) + b1_ref[...],
                  0.0)
  o = jnp.dot(h, w2_ref[...], preferred_element_type=jnp.float32) + b2_ref[...]
  o_ref[...] = jnp.maximum(o, 0.0).astype(jnp.bfloat16)


def _dense_mlp(x, w1, b1, w2, b2):
  return pl.pallas_call(
      _dense_mlp_body,
      out_shape=jax.ShapeDtypeStruct((B, D), jnp.bfloat16),
  )(x, w1, b1.reshape(1, -1), w2, b2.reshape(1, -1))


def _top_mlp_body(dx_ref, emb_ref, w1a_ref, w1b_ref, b1_ref, w2_ref, b2_ref,
                  o_ref):
  h = jnp.dot(dx_ref[...], w1a_ref[...].astype(jnp.bfloat16),
              preferred_element_type=jnp.float32)
  h += jnp.dot(emb_ref[...], w1b_ref[...].astype(jnp.bfloat16),
               preferred_element_type=jnp.float32)
  h = jnp.maximum(h + b1_ref[...], 0.0)
  o = jnp.dot(h, w2_ref[...], preferred_element_type=jnp.float32)
  o_ref[...] = jnp.maximum(o + b2_ref[...], 0.0)


def _top_mlp(dense_x, emb_flat, w1a, w1b, b1, w2_pad, b2):
  blk = 256
  grid = (B // blk,)
  return pl.pallas_call(
      _top_mlp_body,
      grid=grid,
      in_specs=[
          pl.BlockSpec((blk, D), lambda i: (i, 0)),
          pl.BlockSpec((blk, T * D), lambda i: (i, 0)),
          pl.BlockSpec((D, HID), lambda i: (0, 0)),
          pl.BlockSpec((T * D, HID), lambda i: (0, 0)),
          pl.BlockSpec((1, HID), lambda i: (0, 0)),
          pl.BlockSpec((HID, 128), lambda i: (0, 0)),
          pl.BlockSpec((1, 1), lambda i: (0, 0)),
      ],
      out_specs=pl.BlockSpec((blk, 128), lambda i: (i, 0)),
      out_shape=jax.ShapeDtypeStruct((B, 128), jnp.float32),
  )(dense_x, emb_flat, w1a, w1b, b1.reshape(1, -1), w2_pad, b2.reshape(1, 1))


def kernel(dense_features, sharded_sparse_features, tables,
           d_W1, d_b1, d_W2, d_b2, o_W1, o_b1, o_W2, o_b2):
  offs = (jnp.arange(T, dtype=jnp.int32) * V)[None, :, None]
  idx = (sharded_sparse_features.astype(jnp.int32) + offs).reshape(
      TOTAL_CHUNKS, 1, CHUNK)
  table_flat = tables.reshape(T * V, D)
  sidx = jnp.asarray(_SIDX)
  zeros_blk = jnp.zeros((CHUNK, D), jnp.float32)

  emb = _sc_embedding_bags(table_flat, idx, sidx, zeros_blk)
  dense_x = _dense_mlp(dense_features, d_W1, d_b1, d_W2, d_b2)
  emb_bf = emb.reshape(B, T * D).astype(jnp.bfloat16)
  w2_pad = jnp.pad(o_W2, ((0, 0), (0, 127)))
  logits_pad = _top_mlp(dense_x, emb_bf,
                        o_W1[:D], o_W1[D:], o_b1, w2_pad, o_b2)
  return logits_pad[:, :1]
